# baseline (speedup 1.0000x reference)
import functools

import jax
import jax.numpy as jnp
from jax import lax
from jax.experimental import pallas as pl
from jax.experimental.pallas import tpu as pltpu
from jax.experimental.pallas import tpu_sc as plsc

N = 10000
NPAD = 10240
D = 128
E = 320000
C = 128
NSUB = 16
NCORE = 2
NCH = 160
EPAD = NSUB * NCH * C
IB = 16
RPT = NPAD // NSUB
ZR = 16

BLK = 512
NB = NPAD // BLK
_HIGHEST = jax.lax.Precision.HIGHEST


def _make_sc_agg(with_deg: bool):
    mesh = plsc.VectorSubcoreMesh(core_axis_name="c", subcore_axis_name="s")
    out_types = [jax.ShapeDtypeStruct((NCORE * NPAD, D), jnp.float32)]
    scratch = [
        pltpu.VMEM((IB, C), jnp.int32),
        pltpu.VMEM((IB, C), jnp.int32),
        pltpu.VMEM((C, D), jnp.float32),
        pltpu.VMEM((C, D), jnp.float32),
        pltpu.VMEM((ZR, D), jnp.float32),
        pltpu.VMEM_SHARED((NPAD, D), jnp.float32),
        pltpu.SemaphoreType.DMA,
        pltpu.SemaphoreType.DMA,
        pltpu.SemaphoreType.DMA,
        pltpu.SemaphoreType.DMA,
    ]
    if with_deg:
        out_types.append(jax.ShapeDtypeStruct((NPAD,), jnp.float32))
        scratch += [
            pltpu.VMEM((RPT,), jnp.float32),
            pltpu.VMEM((C,), jnp.float32),
            pltpu.VMEM_SHARED((NPAD,), jnp.float32),
        ]

    def body(hs, src2, dst2, aggs, *rest):
        if with_deg:
            (deg_out, src_idx, dst_idx, rows0, rows1, zeros2, acc,
             gsem0, gsem1, ssem0, ssem1, zeros1, ones, deg_sh) = rest
        else:
            (src_idx, dst_idx, rows0, rows1, zeros2, acc,
             gsem0, gsem1, ssem0, ssem1) = rest
        c = lax.axis_index("c")
        s = lax.axis_index("s")
        w = c * NSUB + s

        @pl.loop(0, ZR)
        def _fill(r):
            @pl.loop(0, D // 16)
            def _fill_row(i):
                zeros2[r, pl.ds(i * 16, 16)] = jnp.zeros((16,), jnp.float32)

        @pl.loop(0, RPT // ZR)
        def _zero(k):
            pltpu.sync_copy(zeros2, acc.at[pl.ds(s * RPT + k * ZR, ZR)])

        if with_deg:
            @pl.loop(0, RPT // 16)
            def _fill1(i):
                zeros1[pl.ds(i * 16, 16)] = jnp.zeros((16,), jnp.float32)

            @pl.loop(0, C // 16)
            def _fillo(i):
                ones[pl.ds(i * 16, 16)] = jnp.full((16,), 1.0, jnp.float32)

            @pl.when(c == 0)
            def _zdeg():
                pltpu.sync_copy(zeros1, deg_sh.at[pl.ds(s * RPT, RPT)])

        plsc.subcore_barrier()

        def scat0(j):
            return pltpu.make_async_copy(rows0, acc.at[dst_idx.at[j]], ssem0)

        def scat1(j):
            return pltpu.make_async_copy(rows1, acc.at[dst_idx.at[j]], ssem1)

        @pl.loop(0, NCH // IB)
        def _stage(t):
            @pl.when(t > 0)
            def _drain():
                scat0(0).wait()
                scat1(1).wait()
            pltpu.sync_copy(src2.at[pl.ds(w * NCH + t * IB, IB)], src_idx)
            pltpu.sync_copy(dst2.at[pl.ds(s * NCH + t * IB, IB)], dst_idx)

            @pl.loop(0, IB // 2)
            def _pair(k):
                j0 = 2 * k
                j1 = 2 * k + 1

                @pl.when(k > 0)
                def _reuse():
                    scat0(j0).wait()
                    scat1(j1).wait()

                g0 = pltpu.make_async_copy(hs.at[src_idx.at[j0]], rows0, gsem0)
                g1 = pltpu.make_async_copy(hs.at[src_idx.at[j1]], rows1, gsem1)
                g0.start()
                g1.start()
                g0.wait()
                scat0(j0).start(add=True)
                g1.wait()
                scat1(j1).start(add=True)

            if with_deg:
                @pl.when(c == 0)
                def _degt():
                    @pl.loop(0, IB)
                    def _degj(j):
                        pltpu.sync_copy(ones, deg_sh.at[dst_idx.at[j]], add=True)

        scat0(0).wait()
        scat1(1).wait()
        plsc.subcore_barrier()

        pltpu.sync_copy(acc.at[pl.ds(s * RPT, RPT)],
                        aggs.at[pl.ds(c * NPAD + s * RPT, RPT)])
        if with_deg:
            @pl.when(c == 0)
            def _wdeg():
                pltpu.sync_copy(deg_sh.at[pl.ds(s * RPT, RPT)],
                                deg_out.at[pl.ds(s * RPT, RPT)])

    return pl.kernel(body, out_type=tuple(out_types) if with_deg else out_types[0],
                     mesh=mesh, scratch_types=scratch)


_sc_agg_deg = _make_sc_agg(with_deg=True)
_sc_agg = _make_sc_agg(with_deg=False)


def _mm_body(x_ref, w_ref, b_ref, h_ref):
    h = lax.dot_general(x_ref[...], w_ref[0], (((1,), (0,)), ((), ())),
                        precision=_HIGHEST, preferred_element_type=jnp.float32)
    h_ref[...] = h + b_ref[0, 0]


def _tc_mm(xs, Ws, bs):
    return pl.pallas_call(
        _mm_body,
        grid=(NCORE * NB,),
        in_specs=[
            pl.BlockSpec((BLK, D), lambda i: (i, 0)),
            pl.BlockSpec((1, D, D), lambda i: (i // NB, 0, 0)),
            pl.BlockSpec((1, 1, D), lambda i: (i // NB, 0, 0)),
        ],
        out_specs=pl.BlockSpec((BLK, D), lambda i: (i, 0)),
        out_shape=jax.ShapeDtypeStruct((NCORE * NPAD, D), jnp.float32),
    )(xs, Ws, bs)


def _combine(a_ref, o_ref, deg_ref, dinv_ref):
    dg = jnp.maximum(deg_ref[...], 1.0)
    return jax.nn.relu(a_ref[...] / dg + o_ref[...] * dinv_ref[...])


def _cmm_body(a_ref, o_ref, deg_ref, dinv_ref, w_ref, b_ref, xc_ref, h_ref):
    xc = _combine(a_ref, o_ref, deg_ref, dinv_ref)
    xc_ref[...] = xc
    h = lax.dot_general(xc, w_ref[0], (((1,), (0,)), ((), ())),
                        precision=_HIGHEST, preferred_element_type=jnp.float32)
    h_ref[...] = h + b_ref[0, 0]


def _tc_combine_mm(aggs, deg, dinv, Ws, bs):
    return pl.pallas_call(
        _cmm_body,
        grid=(NCORE * NB,),
        in_specs=[
            pl.BlockSpec((BLK, D), lambda i: (i, 0)),
            pl.BlockSpec((BLK, D), lambda i: ((i + NB) % (NCORE * NB), 0)),
            pl.BlockSpec((BLK, 1), lambda i: (i % NB, 0)),
            pl.BlockSpec((BLK, 1), lambda i: (i % NB, 0)),
            pl.BlockSpec((1, D, D), lambda i: (i // NB, 0, 0)),
            pl.BlockSpec((1, 1, D), lambda i: (i // NB, 0, 0)),
        ],
        out_specs=[
            pl.BlockSpec((BLK, D), lambda i: (i, 0)),
            pl.BlockSpec((BLK, D), lambda i: (i, 0)),
        ],
        out_shape=[
            jax.ShapeDtypeStruct((NCORE * NPAD, D), jnp.float32),
            jax.ShapeDtypeStruct((NCORE * NPAD, D), jnp.float32),
        ],
    )(aggs, aggs, deg, dinv, Ws, bs)


def _c_body(a_ref, o_ref, deg_ref, dinv_ref, xc_ref):
    xc_ref[...] = _combine(a_ref, o_ref, deg_ref, dinv_ref)


def _tc_combine(aggs, deg, dinv):
    return pl.pallas_call(
        _c_body,
        grid=(NCORE * NB,),
        in_specs=[
            pl.BlockSpec((BLK, D), lambda i: (i, 0)),
            pl.BlockSpec((BLK, D), lambda i: ((i + NB) % (NCORE * NB), 0)),
            pl.BlockSpec((BLK, 1), lambda i: (i % NB, 0)),
            pl.BlockSpec((BLK, 1), lambda i: (i % NB, 0)),
        ],
        out_specs=pl.BlockSpec((BLK, D), lambda i: (i, 0)),
        out_shape=jax.ShapeDtypeStruct((NCORE * NPAD, D), jnp.float32),
    )(aggs, aggs, deg, dinv)


def kernel(x, view2, edge_index, D_inv,
           W1, b1, W2, b2, W3, b3, W4, b4, W5, b5, W6, b6):
    padn = ((0, NPAD - N), (0, 0))
    xs = jnp.concatenate([jnp.pad(x, padn), jnp.pad(view2, padn)], axis=0)
    dinv = jnp.pad(D_inv, padn)

    src, dst = lax.sort_key_val(edge_index[0], edge_index[1])
    pad_e = EPAD - E
    src_p = jnp.concatenate([src, jnp.zeros((pad_e,), jnp.int32)])
    dst_p = jnp.concatenate(
        [dst, N + (jnp.arange(pad_e, dtype=jnp.int32) % (NPAD - N))])
    blocks = src_p.reshape(NSUB, NCH, C)
    src3 = jnp.concatenate([blocks, blocks + NPAD], axis=0)
    src3 = src3.reshape(NCORE * NSUB * NCH, C)
    dst3 = dst_p.reshape(NSUB * NCH, C)

    W14 = jnp.stack([W1, W4])
    b14 = jnp.stack([b1, b4]).reshape(NCORE, 1, D)
    W25 = jnp.stack([W2, W5])
    b25 = jnp.stack([b2, b5]).reshape(NCORE, 1, D)
    W36 = jnp.stack([W3, W6])
    b36 = jnp.stack([b3, b6]).reshape(NCORE, 1, D)

    h = _tc_mm(xs, W14, b14)
    agg, deg = _sc_agg_deg(h, src3, dst3)
    deg = deg.reshape(NPAD, 1)
    xc1, h = _tc_combine_mm(agg, deg, dinv, W25, b25)
    agg = _sc_agg(h, src3, dst3)
    xc2, h = _tc_combine_mm(agg, deg, dinv, W36, b36)
    agg = _sc_agg(h, src3, dst3)
    xc3 = _tc_combine(agg, deg, dinv)

    q = jnp.concatenate([xc1[:N], xc2[:N], xc3[:N]], axis=1)
    p = jnp.concatenate([xc1[NPAD:NPAD + N], xc2[NPAD:NPAD + N],
                         xc3[NPAD:NPAD + N]], axis=1)
    return (q, p)


# speedup vs baseline: 1.9486x; 1.9486x over previous
import jax
import jax.numpy as jnp
from jax import lax
from jax.experimental import pallas as pl
from jax.experimental.pallas import tpu as pltpu
from jax.experimental.pallas import tpu_sc as plsc

N = 10000
NPAD = 10240
D = 128
DH = D // 2
E = 320000
C = 128
NSUB = 16
NCORE = 2
NCH = 160
EPAD = NSUB * NCH * C
IB = 16
RPT = NPAD // NSUB
ZR = 32

BLK = 512
NB = NPAD // BLK
_HIGHEST = jax.lax.Precision.HIGHEST


def _make_sc_agg(with_deg: bool):
    mesh = plsc.VectorSubcoreMesh(core_axis_name="c", subcore_axis_name="s")
    out_types = [jax.ShapeDtypeStruct((NCORE * NPAD, D), jnp.float32)]
    scratch = [
        pltpu.VMEM((IB, C), jnp.int32),
        pltpu.VMEM((IB, C), jnp.int32),
        pltpu.VMEM((C, DH), jnp.float32),
        pltpu.VMEM((C, DH), jnp.float32),
        pltpu.VMEM((ZR, DH), jnp.float32),
        pltpu.VMEM_SHARED((NPAD, DH), jnp.float32),
        pltpu.VMEM_SHARED((NPAD, DH), jnp.float32),
        pltpu.SemaphoreType.DMA,
        pltpu.SemaphoreType.DMA,
        pltpu.SemaphoreType.DMA,
        pltpu.SemaphoreType.DMA,
    ]
    if with_deg:
        out_types.append(jax.ShapeDtypeStruct((NPAD,), jnp.float32))
        scratch += [
            pltpu.VMEM((RPT,), jnp.float32),
            pltpu.VMEM((C,), jnp.float32),
            pltpu.VMEM_SHARED((NPAD,), jnp.float32),
        ]

    def body(hs, src2, dst2, aggs, *rest):
        if with_deg:
            (deg_out, src_idx, dst_idx, rows0, rows1, zeros2, table, acc,
             gsem0, gsem1, ssem0, ssem1, zeros1, ones, deg_sh) = rest
        else:
            (src_idx, dst_idx, rows0, rows1, zeros2, table, acc,
             gsem0, gsem1, ssem0, ssem1) = rest
        c = lax.axis_index("c")
        s = lax.axis_index("s")

        @pl.loop(0, ZR)
        def _fill(r):
            @pl.loop(0, DH // 16)
            def _fill_row(i):
                zeros2[r, pl.ds(i * 16, 16)] = jnp.zeros((16,), jnp.float32)

        if with_deg:
            @pl.loop(0, RPT // 16)
            def _fill1(i):
                zeros1[pl.ds(i * 16, 16)] = jnp.zeros((16,), jnp.float32)

            @pl.loop(0, C // 16)
            def _fillo(i):
                ones[pl.ds(i * 16, 16)] = jnp.full((16,), 1.0, jnp.float32)

            @pl.when(c == 0)
            def _zdeg():
                pltpu.sync_copy(zeros1, deg_sh.at[pl.ds(s * RPT, RPT)])

        def scat0(j):
            return pltpu.make_async_copy(rows0, acc.at[dst_idx.at[j]], ssem0)

        def scat1(j):
            return pltpu.make_async_copy(rows1, acc.at[dst_idx.at[j]], ssem1)

        for p in range(2):
            @pl.loop(0, RPT // ZR)
            def _zero(k):
                pltpu.sync_copy(zeros2, acc.at[pl.ds(s * RPT + k * ZR, ZR)])

            pltpu.sync_copy(
                hs.at[pl.ds(c * NPAD + s * RPT, RPT), pl.ds(p * DH, DH)],
                table.at[pl.ds(s * RPT, RPT)])
            plsc.subcore_barrier()

            @pl.loop(0, NCH // IB)
            def _stage(t):
                @pl.when(t > 0)
                def _drain():
                    scat0(0).wait()
                    scat1(1).wait()
                pltpu.sync_copy(src2.at[pl.ds(s * NCH + t * IB, IB)], src_idx)
                pltpu.sync_copy(dst2.at[pl.ds(s * NCH + t * IB, IB)], dst_idx)

                @pl.loop(0, IB // 2)
                def _pair(k):
                    j0 = 2 * k
                    j1 = 2 * k + 1

                    @pl.when(k > 0)
                    def _reuse():
                        scat0(j0).wait()
                        scat1(j1).wait()

                    g0 = pltpu.make_async_copy(table.at[src_idx.at[j0]],
                                               rows0, gsem0)
                    g1 = pltpu.make_async_copy(table.at[src_idx.at[j1]],
                                               rows1, gsem1)
                    g0.start()
                    g1.start()
                    g0.wait()
                    scat0(j0).start(add=True)
                    g1.wait()
                    scat1(j1).start(add=True)

                if with_deg and p == 0:
                    @pl.when(c == 0)
                    def _degt():
                        @pl.loop(0, IB)
                        def _degj(j):
                            pltpu.sync_copy(ones, deg_sh.at[dst_idx.at[j]],
                                            add=True)

            scat0(0).wait()
            scat1(1).wait()
            plsc.subcore_barrier()
            pltpu.sync_copy(
                acc.at[pl.ds(s * RPT, RPT)],
                aggs.at[pl.ds(c * NPAD + s * RPT, RPT), pl.ds(p * DH, DH)])
            plsc.subcore_barrier()

        if with_deg:
            @pl.when(c == 0)
            def _wdeg():
                pltpu.sync_copy(deg_sh.at[pl.ds(s * RPT, RPT)],
                                deg_out.at[pl.ds(s * RPT, RPT)])

    return pl.kernel(body, out_type=tuple(out_types) if with_deg else out_types[0],
                     mesh=mesh, scratch_types=scratch,
                     compiler_params=pltpu.CompilerParams(
                         use_tc_tiling_on_sc=False))


_sc_agg_deg = _make_sc_agg(with_deg=True)
_sc_agg = _make_sc_agg(with_deg=False)


def _mm_body(x_ref, w_ref, b_ref, h_ref):
    h = lax.dot_general(x_ref[...], w_ref[0], (((1,), (0,)), ((), ())),
                        precision=_HIGHEST, preferred_element_type=jnp.float32)
    h_ref[...] = h + b_ref[0, 0]


def _tc_mm(xs, Ws, bs):
    return pl.pallas_call(
        _mm_body,
        grid=(NCORE * NB,),
        in_specs=[
            pl.BlockSpec((BLK, D), lambda i: (i, 0)),
            pl.BlockSpec((1, D, D), lambda i: (i // NB, 0, 0)),
            pl.BlockSpec((1, 1, D), lambda i: (i // NB, 0, 0)),
        ],
        out_specs=pl.BlockSpec((BLK, D), lambda i: (i, 0)),
        out_shape=jax.ShapeDtypeStruct((NCORE * NPAD, D), jnp.float32),
    )(xs, Ws, bs)


def _combine(a_ref, o_ref, deg_ref, dinv_ref):
    dg = jnp.maximum(deg_ref[...], 1.0)
    return jax.nn.relu(a_ref[...] / dg + o_ref[...] * dinv_ref[...])


def _cmm_body(a_ref, o_ref, deg_ref, dinv_ref, w_ref, b_ref, xc_ref, h_ref):
    xc = _combine(a_ref, o_ref, deg_ref, dinv_ref)
    xc_ref[...] = xc
    h = lax.dot_general(xc, w_ref[0], (((1,), (0,)), ((), ())),
                        precision=_HIGHEST, preferred_element_type=jnp.float32)
    h_ref[...] = h + b_ref[0, 0]


def _tc_combine_mm(aggs, deg, dinv, Ws, bs):
    return pl.pallas_call(
        _cmm_body,
        grid=(NCORE * NB,),
        in_specs=[
            pl.BlockSpec((BLK, D), lambda i: (i, 0)),
            pl.BlockSpec((BLK, D), lambda i: ((i + NB) % (NCORE * NB), 0)),
            pl.BlockSpec((BLK, 1), lambda i: (i % NB, 0)),
            pl.BlockSpec((BLK, 1), lambda i: (i % NB, 0)),
            pl.BlockSpec((1, D, D), lambda i: (i // NB, 0, 0)),
            pl.BlockSpec((1, 1, D), lambda i: (i // NB, 0, 0)),
        ],
        out_specs=[
            pl.BlockSpec((BLK, D), lambda i: (i, 0)),
            pl.BlockSpec((BLK, D), lambda i: (i, 0)),
        ],
        out_shape=[
            jax.ShapeDtypeStruct((NCORE * NPAD, D), jnp.float32),
            jax.ShapeDtypeStruct((NCORE * NPAD, D), jnp.float32),
        ],
    )(aggs, aggs, deg, dinv, Ws, bs)


def _c_body(a_ref, o_ref, deg_ref, dinv_ref, xc_ref):
    xc_ref[...] = _combine(a_ref, o_ref, deg_ref, dinv_ref)


def _tc_combine(aggs, deg, dinv):
    return pl.pallas_call(
        _c_body,
        grid=(NCORE * NB,),
        in_specs=[
            pl.BlockSpec((BLK, D), lambda i: (i, 0)),
            pl.BlockSpec((BLK, D), lambda i: ((i + NB) % (NCORE * NB), 0)),
            pl.BlockSpec((BLK, 1), lambda i: (i % NB, 0)),
            pl.BlockSpec((BLK, 1), lambda i: (i % NB, 0)),
        ],
        out_specs=pl.BlockSpec((BLK, D), lambda i: (i, 0)),
        out_shape=jax.ShapeDtypeStruct((NCORE * NPAD, D), jnp.float32),
    )(aggs, aggs, deg, dinv)


def kernel(x, view2, edge_index, D_inv,
           W1, b1, W2, b2, W3, b3, W4, b4, W5, b5, W6, b6):
    padn = ((0, NPAD - N), (0, 0))
    xs = jnp.concatenate([jnp.pad(x, padn), jnp.pad(view2, padn)], axis=0)
    dinv = jnp.pad(D_inv, padn)

    src, dst = edge_index[0], edge_index[1]
    pad_e = EPAD - E
    src_p = jnp.concatenate([src, jnp.zeros((pad_e,), jnp.int32)])
    dst_p = jnp.concatenate(
        [dst, N + (jnp.arange(pad_e, dtype=jnp.int32) % (NPAD - N))])
    src2 = src_p.reshape(NSUB * NCH, C)
    dst2 = dst_p.reshape(NSUB * NCH, C)

    W14 = jnp.stack([W1, W4])
    b14 = jnp.stack([b1, b4]).reshape(NCORE, 1, D)
    W25 = jnp.stack([W2, W5])
    b25 = jnp.stack([b2, b5]).reshape(NCORE, 1, D)
    W36 = jnp.stack([W3, W6])
    b36 = jnp.stack([b3, b6]).reshape(NCORE, 1, D)

    h = _tc_mm(xs, W14, b14)
    agg, deg = _sc_agg_deg(h, src2, dst2)
    deg = deg.reshape(NPAD, 1)
    xc1, h = _tc_combine_mm(agg, deg, dinv, W25, b25)
    agg = _sc_agg(h, src2, dst2)
    xc2, h = _tc_combine_mm(agg, deg, dinv, W36, b36)
    agg = _sc_agg(h, src2, dst2)
    xc3 = _tc_combine(agg, deg, dinv)

    q = jnp.concatenate([xc1[:N], xc2[:N], xc3[:N]], axis=1)
    p = jnp.concatenate([xc1[NPAD:NPAD + N], xc2[NPAD:NPAD + N],
                         xc3[NPAD:NPAD + N]], axis=1)
    return (q, p)


# speedup vs baseline: 2.0167x; 1.0350x over previous
import jax
import jax.numpy as jnp
from jax import lax
from jax.experimental import pallas as pl
from jax.experimental.pallas import tpu as pltpu
from jax.experimental.pallas import tpu_sc as plsc

N = 10000
NPAD = 10240
D = 128
DH = D // 2
E = 320000
C = 128
NSUB = 16
NCORE = 2
NCH = 160
EPAD = NSUB * NCH * C
IB = 80
RPT = NPAD // NSUB
ZR = 32

BLK = 512
NB = NPAD // BLK
_HIGHEST = jax.lax.Precision.HIGHEST


def _make_sc_agg(with_deg: bool):
    mesh = plsc.VectorSubcoreMesh(core_axis_name="c", subcore_axis_name="s")
    out_types = [jax.ShapeDtypeStruct((NCORE * NPAD, D), jnp.float32)]
    scratch = [
        pltpu.VMEM((IB, C), jnp.int32),
        pltpu.VMEM((IB, C), jnp.int32),
        pltpu.VMEM((C, DH), jnp.float32),
        pltpu.VMEM((C, DH), jnp.float32),
        pltpu.VMEM((ZR, DH), jnp.float32),
        pltpu.VMEM_SHARED((NPAD, DH), jnp.float32),
        pltpu.VMEM_SHARED((NPAD, DH), jnp.float32),
        pltpu.SemaphoreType.DMA,
        pltpu.SemaphoreType.DMA,
        pltpu.SemaphoreType.DMA,
        pltpu.SemaphoreType.DMA,
    ]
    if with_deg:
        out_types.append(jax.ShapeDtypeStruct((NPAD,), jnp.float32))
        scratch += [
            pltpu.VMEM((RPT,), jnp.float32),
            pltpu.VMEM((C,), jnp.float32),
            pltpu.VMEM_SHARED((NPAD,), jnp.float32),
        ]

    def body(hs, src2, dst2, aggs, *rest):
        if with_deg:
            (deg_out, src_idx, dst_idx, rows0, rows1, zeros2, table, acc,
             gsem0, gsem1, ssem0, ssem1, zeros1, ones, deg_sh) = rest
        else:
            (src_idx, dst_idx, rows0, rows1, zeros2, table, acc,
             gsem0, gsem1, ssem0, ssem1) = rest
        c = lax.axis_index("c")
        s = lax.axis_index("s")

        @pl.loop(0, ZR)
        def _fill(r):
            @pl.loop(0, DH // 16)
            def _fill_row(i):
                zeros2[r, pl.ds(i * 16, 16)] = jnp.zeros((16,), jnp.float32)

        if with_deg:
            @pl.loop(0, RPT // 16)
            def _fill1(i):
                zeros1[pl.ds(i * 16, 16)] = jnp.zeros((16,), jnp.float32)

            @pl.loop(0, C // 16)
            def _fillo(i):
                ones[pl.ds(i * 16, 16)] = jnp.full((16,), 1.0, jnp.float32)

            @pl.when(c == 0)
            def _zdeg():
                pltpu.sync_copy(zeros1, deg_sh.at[pl.ds(s * RPT, RPT)])

        def scat0(j):
            return pltpu.make_async_copy(rows0, acc.at[dst_idx.at[j]], ssem0)

        def scat1(j):
            return pltpu.make_async_copy(rows1, acc.at[dst_idx.at[j]], ssem1)

        for p in range(2):
            @pl.loop(0, RPT // ZR)
            def _zero(k):
                pltpu.sync_copy(zeros2, acc.at[pl.ds(s * RPT + k * ZR, ZR)])

            pltpu.sync_copy(
                hs.at[pl.ds(c * NPAD + s * RPT, RPT), pl.ds(p * DH, DH)],
                table.at[pl.ds(s * RPT, RPT)])
            plsc.subcore_barrier()

            @pl.loop(0, NCH // IB)
            def _stage(t):
                @pl.when(t > 0)
                def _drain():
                    scat0(0).wait()
                    scat1(1).wait()
                pltpu.sync_copy(src2.at[pl.ds(s * NCH + t * IB, IB)], src_idx)
                pltpu.sync_copy(dst2.at[pl.ds(s * NCH + t * IB, IB)], dst_idx)

                @pl.loop(0, IB // 2)
                def _pair(k):
                    j0 = 2 * k
                    j1 = 2 * k + 1

                    @pl.when(k > 0)
                    def _reuse():
                        scat0(j0).wait()
                        scat1(j1).wait()

                    g0 = pltpu.make_async_copy(table.at[src_idx.at[j0]],
                                               rows0, gsem0)
                    g1 = pltpu.make_async_copy(table.at[src_idx.at[j1]],
                                               rows1, gsem1)
                    g0.start()
                    g1.start()
                    g0.wait()
                    scat0(j0).start(add=True)
                    g1.wait()
                    scat1(j1).start(add=True)

                if with_deg and p == 0:
                    @pl.when(c == 0)
                    def _degt():
                        @pl.loop(0, IB)
                        def _degj(j):
                            pltpu.sync_copy(ones, deg_sh.at[dst_idx.at[j]],
                                            add=True)

            scat0(0).wait()
            scat1(1).wait()
            plsc.subcore_barrier()
            pltpu.sync_copy(
                acc.at[pl.ds(s * RPT, RPT)],
                aggs.at[pl.ds(c * NPAD + s * RPT, RPT), pl.ds(p * DH, DH)])

        if with_deg:
            @pl.when(c == 0)
            def _wdeg():
                pltpu.sync_copy(deg_sh.at[pl.ds(s * RPT, RPT)],
                                deg_out.at[pl.ds(s * RPT, RPT)])

    return pl.kernel(body, out_type=tuple(out_types) if with_deg else out_types[0],
                     mesh=mesh, scratch_types=scratch,
                     compiler_params=pltpu.CompilerParams(
                         use_tc_tiling_on_sc=False))


_sc_agg_deg = _make_sc_agg(with_deg=True)
_sc_agg = _make_sc_agg(with_deg=False)


def _mm_body(x_ref, w_ref, b_ref, h_ref):
    h = lax.dot_general(x_ref[...], w_ref[0], (((1,), (0,)), ((), ())),
                        precision=_HIGHEST, preferred_element_type=jnp.float32)
    h_ref[...] = h + b_ref[0, 0]


def _tc_mm(xs, Ws, bs):
    return pl.pallas_call(
        _mm_body,
        grid=(NCORE * NB,),
        in_specs=[
            pl.BlockSpec((BLK, D), lambda i: (i, 0)),
            pl.BlockSpec((1, D, D), lambda i: (i // NB, 0, 0)),
            pl.BlockSpec((1, 1, D), lambda i: (i // NB, 0, 0)),
        ],
        out_specs=pl.BlockSpec((BLK, D), lambda i: (i, 0)),
        out_shape=jax.ShapeDtypeStruct((NCORE * NPAD, D), jnp.float32),
    )(xs, Ws, bs)


def _combine(a_ref, o_ref, deg_ref, dinv_ref):
    dg = jnp.maximum(deg_ref[...], 1.0)
    return jax.nn.relu(a_ref[...] / dg + o_ref[...] * dinv_ref[...])


def _cmm_body(a_ref, o_ref, deg_ref, dinv_ref, w_ref, b_ref, xc_ref, h_ref):
    xc = _combine(a_ref, o_ref, deg_ref, dinv_ref)
    xc_ref[...] = xc
    h = lax.dot_general(xc, w_ref[0], (((1,), (0,)), ((), ())),
                        precision=_HIGHEST, preferred_element_type=jnp.float32)
    h_ref[...] = h + b_ref[0, 0]


def _tc_combine_mm(aggs, deg, dinv, Ws, bs):
    return pl.pallas_call(
        _cmm_body,
        grid=(NCORE * NB,),
        in_specs=[
            pl.BlockSpec((BLK, D), lambda i: (i, 0)),
            pl.BlockSpec((BLK, D), lambda i: ((i + NB) % (NCORE * NB), 0)),
            pl.BlockSpec((BLK, 1), lambda i: (i % NB, 0)),
            pl.BlockSpec((BLK, 1), lambda i: (i % NB, 0)),
            pl.BlockSpec((1, D, D), lambda i: (i // NB, 0, 0)),
            pl.BlockSpec((1, 1, D), lambda i: (i // NB, 0, 0)),
        ],
        out_specs=[
            pl.BlockSpec((BLK, D), lambda i: (i, 0)),
            pl.BlockSpec((BLK, D), lambda i: (i, 0)),
        ],
        out_shape=[
            jax.ShapeDtypeStruct((NCORE * NPAD, D), jnp.float32),
            jax.ShapeDtypeStruct((NCORE * NPAD, D), jnp.float32),
        ],
    )(aggs, aggs, deg, dinv, Ws, bs)


def _c_body(a_ref, o_ref, deg_ref, dinv_ref, xc_ref):
    xc_ref[...] = _combine(a_ref, o_ref, deg_ref, dinv_ref)


def _tc_combine(aggs, deg, dinv):
    return pl.pallas_call(
        _c_body,
        grid=(NCORE * NB,),
        in_specs=[
            pl.BlockSpec((BLK, D), lambda i: (i, 0)),
            pl.BlockSpec((BLK, D), lambda i: ((i + NB) % (NCORE * NB), 0)),
            pl.BlockSpec((BLK, 1), lambda i: (i % NB, 0)),
            pl.BlockSpec((BLK, 1), lambda i: (i % NB, 0)),
        ],
        out_specs=pl.BlockSpec((BLK, D), lambda i: (i, 0)),
        out_shape=jax.ShapeDtypeStruct((NCORE * NPAD, D), jnp.float32),
    )(aggs, aggs, deg, dinv)


def kernel(x, view2, edge_index, D_inv,
           W1, b1, W2, b2, W3, b3, W4, b4, W5, b5, W6, b6):
    padn = ((0, NPAD - N), (0, 0))
    xs = jnp.concatenate([jnp.pad(x, padn), jnp.pad(view2, padn)], axis=0)
    dinv = jnp.pad(D_inv, padn)

    src, dst = edge_index[0], edge_index[1]
    pad_e = EPAD - E
    src_p = jnp.concatenate([src, jnp.zeros((pad_e,), jnp.int32)])
    dst_p = jnp.concatenate(
        [dst, N + (jnp.arange(pad_e, dtype=jnp.int32) % (NPAD - N))])
    src2 = src_p.reshape(NSUB * NCH, C)
    dst2 = dst_p.reshape(NSUB * NCH, C)

    W14 = jnp.stack([W1, W4])
    b14 = jnp.stack([b1, b4]).reshape(NCORE, 1, D)
    W25 = jnp.stack([W2, W5])
    b25 = jnp.stack([b2, b5]).reshape(NCORE, 1, D)
    W36 = jnp.stack([W3, W6])
    b36 = jnp.stack([b3, b6]).reshape(NCORE, 1, D)

    h = _tc_mm(xs, W14, b14)
    agg, deg = _sc_agg_deg(h, src2, dst2)
    deg = deg.reshape(NPAD, 1)
    xc1, h = _tc_combine_mm(agg, deg, dinv, W25, b25)
    agg = _sc_agg(h, src2, dst2)
    xc2, h = _tc_combine_mm(agg, deg, dinv, W36, b36)
    agg = _sc_agg(h, src2, dst2)
    xc3 = _tc_combine(agg, deg, dinv)

    q = jnp.concatenate([xc1[:N], xc2[:N], xc3[:N]], axis=1)
    p = jnp.concatenate([xc1[NPAD:NPAD + N], xc2[NPAD:NPAD + N],
                         xc3[NPAD:NPAD + N]], axis=1)
    return (q, p)
